# baseline (speedup 1.0000x reference)
import functools

import jax
import jax.numpy as jnp
from jax.experimental import pallas as pl
from jax.experimental.pallas import tpu as pltpu

_MAX_EXP = 15
_NVAL = 16
_NPOS = 16
_EMB = 32
_IN = _NPOS * _EMB
_HID = 256
_ACT = 4
_OH = _NPOS * _NVAL


def _body(bs, boards_ref, table_ref, w1_ref, b1_ref, w2_ref, b2_ref,
          w3_ref, b3_ref, out_ref, u_ref):
    @pl.when(pl.program_id(0) == 0)
    def _():
        t = table_ref[...]
        for p in range(_NPOS):
            u_ref[p * _NVAL:(p + 1) * _NVAL, :] = jnp.dot(
                t, w1_ref[p * _EMB:(p + 1) * _EMB, :],
                preferred_element_type=jnp.float32).astype(jnp.bfloat16)

    enc = jnp.clip(boards_ref[...], 0, _MAX_EXP).astype(jnp.bfloat16)

    rows = jax.lax.broadcasted_iota(jnp.int32, (_NPOS, _OH), 0)
    cols = jax.lax.broadcasted_iota(jnp.int32, (_NPOS, _OH), 1)
    rep = (cols // _NVAL == rows).astype(jnp.bfloat16)
    encrep = jnp.dot(enc, rep, preferred_element_type=jnp.float32)

    vmod = (jax.lax.broadcasted_iota(jnp.int32, (bs, _OH), 1) % _NVAL
            ).astype(jnp.float32)
    oh = (encrep == vmod).astype(jnp.bfloat16)

    h = jnp.dot(oh, u_ref[...], preferred_element_type=jnp.float32)
    h = jnp.maximum(h + b1_ref[...], 0.0).astype(jnp.bfloat16)
    h = jnp.dot(h, w2_ref[...], preferred_element_type=jnp.float32)
    h = jnp.maximum(h + b2_ref[...], 0.0).astype(jnp.bfloat16)
    out_ref[...] = (jnp.dot(h, w3_ref[...], preferred_element_type=jnp.float32)
                    + b3_ref[...])


@jax.jit
def kernel(boards, table, W1, b1, W2, b2, W3, b3):
    B = boards.shape[0]
    bs = 2048
    grid = B // bs

    out = pl.pallas_call(
        functools.partial(_body, bs),
        grid=(grid,),
        in_specs=[
            pl.BlockSpec((bs, _NPOS), lambda i: (i, 0)),
            pl.BlockSpec((_NVAL, _EMB), lambda i: (0, 0)),
            pl.BlockSpec((_IN, _HID), lambda i: (0, 0)),
            pl.BlockSpec((1, _HID), lambda i: (0, 0)),
            pl.BlockSpec((_HID, _HID), lambda i: (0, 0)),
            pl.BlockSpec((1, _HID), lambda i: (0, 0)),
            pl.BlockSpec((_HID, _ACT), lambda i: (0, 0)),
            pl.BlockSpec((1, _ACT), lambda i: (0, 0)),
        ],
        out_specs=pl.BlockSpec((bs, _ACT), lambda i: (i, 0)),
        out_shape=jax.ShapeDtypeStruct((B, _ACT), jnp.float32),
        scratch_shapes=[pltpu.VMEM((_OH, _HID), jnp.bfloat16)],
        compiler_params=pltpu.CompilerParams(
            dimension_semantics=("arbitrary",)),
    )(boards.astype(jnp.int32), table, W1, b1.reshape(1, _HID),
      W2.astype(jnp.bfloat16), b2.reshape(1, _HID),
      W3.astype(jnp.bfloat16), b3.reshape(1, _ACT))
    return out


# speedup vs baseline: 1.0580x; 1.0580x over previous
import functools

import jax
import jax.numpy as jnp
from jax.experimental import pallas as pl
from jax.experimental.pallas import tpu as pltpu

_MAX_EXP = 15
_NVAL = 16
_NPOS = 16
_EMB = 32
_IN = _NPOS * _EMB
_HID = 256
_ACT = 4
_OH = _NPOS * _NVAL


def _body(bs, boards_ref, table_ref, w1_ref, b1_ref, w2_ref, b2_ref,
          w3_ref, b3_ref, out_ref, u_ref):
    @pl.when(pl.program_id(0) == 0)
    def _():
        t = table_ref[...]
        for p in range(_NPOS):
            u_ref[p * _NVAL:(p + 1) * _NVAL, :] = jnp.dot(
                t, w1_ref[p * _EMB:(p + 1) * _EMB, :],
                preferred_element_type=jnp.float32)

    enc = jnp.clip(boards_ref[...], 0, _MAX_EXP).astype(jnp.float32)

    rows = jax.lax.broadcasted_iota(jnp.int32, (_NPOS, _OH), 0)
    cols = jax.lax.broadcasted_iota(jnp.int32, (_NPOS, _OH), 1)
    rep = (cols // _NVAL == rows).astype(jnp.float32)
    encrep = jnp.dot(enc, rep, preferred_element_type=jnp.float32)

    vmod = (jax.lax.broadcasted_iota(jnp.int32, (bs, _OH), 1) % _NVAL
            ).astype(jnp.float32)
    oh = (encrep == vmod).astype(jnp.float32)

    h = jnp.dot(oh, u_ref[...], preferred_element_type=jnp.float32)
    h = jnp.maximum(h + b1_ref[...][None, :], 0.0)
    h = jnp.dot(h, w2_ref[...], preferred_element_type=jnp.float32)
    h = jnp.maximum(h + b2_ref[...][None, :], 0.0)
    out_ref[...] = (jnp.dot(h, w3_ref[...], preferred_element_type=jnp.float32)
                    + b3_ref[...][None, :])


@jax.jit
def kernel(boards, table, W1, b1, W2, b2, W3, b3):
    B = boards.shape[0]
    bs = 2048
    grid = B // bs

    out = pl.pallas_call(
        functools.partial(_body, bs),
        grid=(grid,),
        in_specs=[
            pl.BlockSpec((bs, _NPOS), lambda i: (i, 0)),
            pl.BlockSpec((_NVAL, _EMB), lambda i: (0, 0)),
            pl.BlockSpec((_IN, _HID), lambda i: (0, 0)),
            pl.BlockSpec((_HID,), lambda i: (0,)),
            pl.BlockSpec((_HID, _HID), lambda i: (0, 0)),
            pl.BlockSpec((_HID,), lambda i: (0,)),
            pl.BlockSpec((_HID, _ACT), lambda i: (0, 0)),
            pl.BlockSpec((_ACT,), lambda i: (0,)),
        ],
        out_specs=pl.BlockSpec((bs, _ACT), lambda i: (i, 0)),
        out_shape=jax.ShapeDtypeStruct((B, _ACT), jnp.float32),
        scratch_shapes=[pltpu.VMEM((_OH, _HID), jnp.float32)],
        compiler_params=pltpu.CompilerParams(
            dimension_semantics=("arbitrary",)),
    )(boards.astype(jnp.int32), table, W1, b1, W2, b2, W3, b3)
    return out


# speedup vs baseline: 1.1130x; 1.0520x over previous
import functools

import jax
import jax.numpy as jnp
from jax.experimental import pallas as pl
from jax.experimental.pallas import tpu as pltpu

_MAX_EXP = 15
_NVAL = 16
_NPOS = 16
_EMB = 32
_IN = _NPOS * _EMB
_HID = 256
_ACT = 4
_OH = _NPOS * _NVAL


def _body(bs, boards_ref, table_ref, w1_ref, b1_ref, w2_ref, b2_ref,
          w3_ref, b3_ref, out_ref, u_ref):
    @pl.when(pl.program_id(0) == 0)
    def _():
        t = table_ref[...]
        for p in range(_NPOS):
            u_ref[p * _NVAL:(p + 1) * _NVAL, :] = jnp.dot(
                t, w1_ref[p * _EMB:(p + 1) * _EMB, :],
                preferred_element_type=jnp.float32)

    enc = jnp.clip(boards_ref[...], 0, _MAX_EXP).astype(jnp.float32)

    rows = jax.lax.broadcasted_iota(jnp.int32, (_NPOS, _OH), 0)
    cols = jax.lax.broadcasted_iota(jnp.int32, (_NPOS, _OH), 1)
    rep = (cols // _NVAL == rows).astype(jnp.float32)
    encrep = jnp.dot(enc, rep, preferred_element_type=jnp.float32)

    vmod = (jax.lax.broadcasted_iota(jnp.int32, (bs, _OH), 1) % _NVAL
            ).astype(jnp.float32)
    oh = (encrep == vmod).astype(jnp.float32)

    h = jnp.dot(oh, u_ref[...], preferred_element_type=jnp.float32)
    h = jnp.maximum(h + b1_ref[...][None, :], 0.0)
    h = jnp.dot(h, w2_ref[...], preferred_element_type=jnp.float32)
    h = jnp.maximum(h + b2_ref[...][None, :], 0.0)
    out_ref[...] = (jnp.dot(h, w3_ref[...], preferred_element_type=jnp.float32)
                    + b3_ref[...][None, :])


@jax.jit
def kernel(boards, table, W1, b1, W2, b2, W3, b3):
    B = boards.shape[0]
    bs = 4096
    grid = B // bs

    out = pl.pallas_call(
        functools.partial(_body, bs),
        grid=(grid,),
        in_specs=[
            pl.BlockSpec((bs, _NPOS), lambda i: (i, 0)),
            pl.BlockSpec((_NVAL, _EMB), lambda i: (0, 0)),
            pl.BlockSpec((_IN, _HID), lambda i: (0, 0)),
            pl.BlockSpec((_HID,), lambda i: (0,)),
            pl.BlockSpec((_HID, _HID), lambda i: (0, 0)),
            pl.BlockSpec((_HID,), lambda i: (0,)),
            pl.BlockSpec((_HID, _ACT), lambda i: (0, 0)),
            pl.BlockSpec((_ACT,), lambda i: (0,)),
        ],
        out_specs=pl.BlockSpec((bs, _ACT), lambda i: (i, 0)),
        out_shape=jax.ShapeDtypeStruct((B, _ACT), jnp.float32),
        scratch_shapes=[pltpu.VMEM((_OH, _HID), jnp.float32)],
        compiler_params=pltpu.CompilerParams(
            dimension_semantics=("arbitrary",)),
    )(boards.astype(jnp.int32), table, W1, b1, W2, b2, W3, b3)
    return out
